# baseline (speedup 1.0000x reference)
import functools

import jax
import jax.numpy as jnp
import numpy as np
from jax.experimental import pallas as pl
from jax.experimental.pallas import tpu as pltpu
from jax.sharding import Mesh, PartitionSpec as P


def _gcn_block(
    xf_ref, xl_ref, ws_ref, wn_ref, b_ref, adj_ref, out_ref, support_ref, *, bm
):
    m = pl.program_id(0)

    @pl.when(m == 0)
    def _():
        support_ref[...] = jnp.dot(
            xf_ref[...], wn_ref[...], preferred_element_type=jnp.float32
        )

    x_blk = xl_ref[pl.ds(m * bm, bm), :]
    acc = jnp.dot(x_blk, ws_ref[...], preferred_element_type=jnp.float32)
    acc += jnp.dot(adj_ref[...], support_ref[...], preferred_element_type=jnp.float32)
    out_ref[...] = acc + b_ref[...]


def _row_block_gcn(x_full, x_local, adj_rows, ws, wn, b2, *, bm):
    n_local, n = adj_rows.shape
    n_full, d_in = x_full.shape
    d_out = ws.shape[1]
    return pl.pallas_call(
        functools.partial(_gcn_block, bm=bm),
        grid=(n_local // bm,),
        in_specs=[
            pl.BlockSpec((n_full, d_in), lambda m: (0, 0)),
            pl.BlockSpec((n_local, d_in), lambda m: (0, 0)),
            pl.BlockSpec((d_in, d_out), lambda m: (0, 0)),
            pl.BlockSpec((d_in, d_out), lambda m: (0, 0)),
            pl.BlockSpec((1, d_out), lambda m: (0, 0)),
            pl.BlockSpec((bm, n), lambda m: (m, 0)),
        ],
        out_specs=pl.BlockSpec((bm, d_out), lambda m: (m, 0)),
        out_shape=jax.ShapeDtypeStruct((n_local, d_out), jnp.float32),
        scratch_shapes=[pltpu.VMEM((n_full, d_out), jnp.float32)],
    )(x_full, x_local, ws, wn, b2, adj_rows)


def _sharded_gcn(x, adj_local, ws, wn, b2, *, bm):
    n_local = adj_local.shape[0]
    idx = jax.lax.axis_index("m")
    x_local = jax.lax.dynamic_slice_in_dim(x, idx * n_local, n_local)
    return _row_block_gcn(x, x_local, adj_local, ws, wn, b2, bm=bm)


def _pick_bm(rows):
    for bm in (400, 200):
        if rows % bm == 0:
            return bm
    return None


def kernel(input, adj, weight_self, weight_neighbor, bias):
    n = adj.shape[0]
    b2 = bias.reshape(1, -1)
    devs = jax.devices()
    nd = len(devs)
    if nd > 1 and n % nd == 0 and _pick_bm(n // nd) is not None:
        bm = _pick_bm(n // nd)
        mesh = Mesh(np.array(devs), ("m",))
        f = jax.shard_map(
            functools.partial(_sharded_gcn, bm=bm),
            mesh=mesh,
            in_specs=(P(), P("m", None), P(), P(), P()),
            out_specs=P("m", None),
            check_vma=False,
        )
        return f(input, adj, weight_self, weight_neighbor, b2)
    return _row_block_gcn(
        input, input, adj, weight_self, weight_neighbor, b2, bm=_pick_bm(n) or 200
    )


# speedup vs baseline: 5.7840x; 5.7840x over previous
import functools

import jax
import jax.numpy as jnp
from jax.experimental import pallas as pl
from jax.experimental.pallas import tpu as pltpu


def _gcn_block(x_ref, ws_ref, wn_ref, b_ref, adj_ref, out_ref, support_ref, *, bm):
    m = pl.program_id(0)

    @pl.when(m == 0)
    def _():
        support_ref[...] = jnp.dot(
            x_ref[...], wn_ref[...], preferred_element_type=jnp.float32
        )

    x_blk = x_ref[pl.ds(m * bm, bm), :]
    acc = jnp.dot(x_blk, ws_ref[...], preferred_element_type=jnp.float32)
    acc += jnp.dot(adj_ref[...], support_ref[...], preferred_element_type=jnp.float32)
    out_ref[...] = acc + b_ref[...]


def kernel(input, adj, weight_self, weight_neighbor, bias):
    n, d_in = input.shape
    d_out = weight_self.shape[1]
    bm = 400
    grid = (n // bm,)
    return pl.pallas_call(
        functools.partial(_gcn_block, bm=bm),
        grid=grid,
        in_specs=[
            pl.BlockSpec((n, d_in), lambda m: (0, 0)),
            pl.BlockSpec((d_in, d_out), lambda m: (0, 0)),
            pl.BlockSpec((d_in, d_out), lambda m: (0, 0)),
            pl.BlockSpec((1, d_out), lambda m: (0, 0)),
            pl.BlockSpec((bm, n), lambda m: (m, 0)),
        ],
        out_specs=pl.BlockSpec((bm, d_out), lambda m: (m, 0)),
        out_shape=jax.ShapeDtypeStruct((n, d_out), jnp.float32),
        scratch_shapes=[pltpu.VMEM((n, d_out), jnp.float32)],
    )(input, weight_self, weight_neighbor, bias.reshape(1, -1), adj)
